# baseline (speedup 1.0000x reference)
import functools
import math

import jax
import jax.numpy as jnp
from jax import lax
from jax.experimental import pallas as pl
from jax.experimental.pallas import tpu as pltpu
from jax.experimental.pallas import tpu_sc as plsc

EPS_BN = 1e-5
GSCALE = 1.0 / math.sqrt(1.0 + EPS_BN)

NSUB = 16
NCORE = 2


def _sc_mesh():
    return plsc.VectorSubcoreMesh(core_axis_name="c", subcore_axis_name="s")


def _sc_degree(col, n):
    e = col.shape[0]
    epc = e // NCORE
    ch = 128
    per = (epc // NSUB) & ~(ch - 1)
    m_lo = per // ch
    m_hi = (epc - per * (NSUB - 1)) // ch
    npt = (n // NSUB) & ~7
    tail = n - npt * NSUB

    zeros_h = jnp.zeros((n, 128), jnp.float32)
    ones_h = jnp.ones((ch, 128), jnp.float32)

    @functools.partial(
        pl.kernel,
        out_type=jax.ShapeDtypeStruct((NCORE, n, 128), jnp.float32),
        mesh=_sc_mesh(),
        scratch_types=[
            pltpu.VMEM((4, ch), jnp.int32),
            pltpu.VMEM((ch, 128), jnp.float32),
            pltpu.VMEM_SHARED((n, 128), jnp.float32),
        ] + [pltpu.SemaphoreType.DMA] * 4,
    )
    def k(col_hbm, zero_hbm, ones_hbm, out_hbm, colv, onesv, accum, s0, s1, i0, i1):
        ssem = [s0, s1]
        isem = [i0, i1]
        c = lax.axis_index("c")
        s = lax.axis_index("s")
        m = jnp.where(s == NSUB - 1, m_hi, m_lo)
        base = c * epc + s * per
        pltpu.sync_copy(col_hbm.at[pl.ds(base, ch)], colv.at[0])
        pltpu.sync_copy(col_hbm.at[pl.ds(base + ch, ch)], colv.at[1])
        pltpu.sync_copy(zero_hbm.at[pl.ds(s * npt, npt)], accum.at[pl.ds(s * npt, npt)])

        @pl.when(s == 0)
        def _():
            pltpu.sync_copy(zero_hbm.at[pl.ds(npt * NSUB, tail)],
                            accum.at[pl.ds(npt * NSUB, tail)])

        pltpu.sync_copy(ones_hbm, onesv)
        plsc.subcore_barrier()

        @pl.loop(0, m_hi, step=2)
        def _(i):
            for b in range(2):
                j = i + b

                @pl.when(j < m)
                def _(j=j, b=b):
                    @pl.when(j >= 2)
                    def _():
                        pltpu.make_async_copy(
                            onesv, accum.at[colv.at[j & 3]], ssem[b]).wait()
                        pltpu.make_async_copy(
                            col_hbm.at[pl.ds(0, ch)], colv.at[j & 3],
                            isem[b]).wait()
                    pltpu.async_copy(onesv, accum.at[colv.at[j & 3]],
                                     ssem[b], add=True)

                @pl.when(j + 2 < m)
                def _(j=j, b=b):
                    pltpu.async_copy(
                        col_hbm.at[pl.ds(base + (j + 2) * ch, ch)],
                        colv.at[(j + 2) & 3], isem[b])

        pltpu.make_async_copy(onesv, accum.at[colv.at[0]], ssem[0]).wait()
        pltpu.make_async_copy(onesv, accum.at[colv.at[1]], ssem[1]).wait()
        plsc.subcore_barrier()
        pltpu.sync_copy(accum.at[pl.ds(s * npt, npt)], out_hbm.at[c, pl.ds(s * npt, npt)])

        @pl.when(s == 0)
        def _():
            pltpu.sync_copy(accum.at[pl.ds(npt * NSUB, tail)],
                            out_hbm.at[c, pl.ds(npt * NSUB, tail)])

    return k(col, zeros_h, ones_h)


_NBUF = 2


def _sc_scatter(xws_flat, rows2, col, n):
    e = col.shape[0]
    ch = 128
    per = (e // NSUB) & ~(ch - 1)
    m_lo = per // ch
    m_hi = (e - per * (NSUB - 1)) // ch
    npt = (n // NSUB) & ~7
    tail = n - npt * NSUB

    zeros128 = jnp.zeros((n, 128), jnp.float32)

    @functools.partial(
        pl.kernel,
        out_type=jax.ShapeDtypeStruct((NCORE, n, 128), jnp.float32),
        mesh=_sc_mesh(),
        scratch_types=[
            pltpu.VMEM((4, ch), jnp.int32),
            pltpu.VMEM((4, ch), jnp.int32),
            pltpu.VMEM((3, ch, 128), jnp.float32),
            pltpu.VMEM_SHARED((n, 128), jnp.float32),
        ] + [pltpu.SemaphoreType.DMA] * 9,
    )
    def k(xws_hbm, rows_hbm, col_hbm, zero_hbm, out_hbm, rowv, colv, msgv, accum,
          g0, g1, g2, s0, s1, s2, i0, i1, i2):
        gsem = [g0, g1, g2]
        ssem = [s0, s1, s2]
        isem = [i0, i1, i2]
        c = lax.axis_index("c")
        s = lax.axis_index("s")
        m = jnp.where(s == NSUB - 1, m_hi, m_lo)
        base = s * per

        def load_idx_sync(j, q):
            pltpu.sync_copy(rows_hbm.at[pl.ds(c * e + base + j * ch, ch)], rowv.at[q])
            pltpu.sync_copy(col_hbm.at[pl.ds(base + j * ch, ch)], colv.at[q])

        load_idx_sync(0, 0)
        load_idx_sync(1, 1)
        load_idx_sync(2, 2)
        pltpu.async_copy(xws_hbm.at[rowv.at[0]], msgv.at[0], gsem[0])
        pltpu.async_copy(xws_hbm.at[rowv.at[1]], msgv.at[1], gsem[1])

        pltpu.sync_copy(zero_hbm.at[pl.ds(s * npt, npt)], accum.at[pl.ds(s * npt, npt)])

        @pl.when(s == 0)
        def _():
            pltpu.sync_copy(zero_hbm.at[pl.ds(npt * NSUB, tail)],
                            accum.at[pl.ds(npt * NSUB, tail)])

        plsc.subcore_barrier()

        @pl.loop(0, m_hi + (3 - m_hi % 3) % 3, step=3)
        def _(i):
            for b in range(3):
                j = i + b
                b2 = (b + 2) % 3

                @pl.when(j < m)
                def _(j=j, b=b):
                    pltpu.make_async_copy(
                        xws_hbm.at[rowv.at[j & 3]], msgv.at[b], gsem[b]).wait()
                    pltpu.async_copy(
                        msgv.at[b], accum.at[colv.at[j & 3]], ssem[b], add=True)

                @pl.when(j + 2 < m)
                def _(j=j, b2=b2):
                    @pl.when(j >= 1)
                    def _():
                        pltpu.make_async_copy(
                            msgv.at[b2], accum.at[colv.at[(j - 1) & 3]],
                            ssem[b2]).wait()
                        pltpu.make_async_copy(
                            rows_hbm.at[pl.ds(0, ch)], rowv.at[(j + 2) & 3],
                            isem[b2]).wait()
                        pltpu.make_async_copy(
                            col_hbm.at[pl.ds(0, ch)], colv.at[(j + 2) & 3],
                            isem[b2]).wait()
                    pltpu.async_copy(
                        xws_hbm.at[rowv.at[(j + 2) & 3]], msgv.at[b2], gsem[b2])

                @pl.when(j + 3 < m)
                def _(j=j, b=b):
                    q = (j + 3) & 3
                    pltpu.async_copy(
                        rows_hbm.at[pl.ds(c * e + base + (j + 3) * ch, ch)],
                        rowv.at[q], isem[b])
                    pltpu.async_copy(
                        col_hbm.at[pl.ds(base + (j + 3) * ch, ch)],
                        colv.at[q], isem[b])

        pltpu.make_async_copy(msgv.at[0], accum.at[colv.at[0]], ssem[0]).wait()
        pltpu.make_async_copy(msgv.at[1], accum.at[colv.at[1]], ssem[1]).wait()
        pltpu.make_async_copy(msgv.at[2], accum.at[colv.at[2]], ssem[2]).wait()
        plsc.subcore_barrier()
        pltpu.sync_copy(accum.at[pl.ds(s * npt, npt)], out_hbm.at[c, pl.ds(s * npt, npt)])

        @pl.when(s == 0)
        def _():
            pltpu.sync_copy(accum.at[pl.ds(npt * NSUB, tail)],
                            out_hbm.at[c, pl.ds(npt * NSUB, tail)])

    return k(xws_flat, rows2, col, zeros128)


def _dinv_block(deg_ref):
    d = deg_ref[0, :, 0:1] + deg_ref[1, :, 0:1] + 1.0
    return lax.rsqrt(d)


def _t0_body(deg_ref, x_ref, w_ref, out_ref):
    d = _dinv_block(deg_ref)
    xw = jnp.dot(x_ref[...], w_ref[...], preferred_element_type=jnp.float32)
    xws = xw * d
    out_ref[0] = xws[:, :128]
    out_ref[1] = xws[:, 128:]


def _tmid_body(deg_ref, s_ref, xp_ref, b_ref, g_ref, be_ref, w_ref, out_ref):
    d = _dinv_block(deg_ref)
    gs = g_ref[...] * GSCALE
    off = b_ref[...] * gs + be_ref[...]
    h0 = jnp.maximum((s_ref[0] + xp_ref[0]) * d * gs[:, :128] + off[:, :128], 0.0)
    h1 = jnp.maximum((s_ref[1] + xp_ref[1]) * d * gs[:, 128:] + off[:, 128:], 0.0)
    res = jnp.dot(h0, w_ref[0:128, :], preferred_element_type=jnp.float32)
    res = res + jnp.dot(h1, w_ref[128:256, :], preferred_element_type=jnp.float32)
    xws = res * d
    out_ref[0] = xws[:, :128]
    out_ref[1] = xws[:, 128:]


def _tlast_body(deg_ref, s_ref, xp_ref, b_ref, g_ref, be_ref, wc_ref, bc_ref, out_ref):
    d = _dinv_block(deg_ref)
    gs = g_ref[...] * GSCALE
    off = b_ref[...] * gs + be_ref[...]
    h0 = jnp.maximum((s_ref[0] + xp_ref[0]) * d * gs[:, :128] + off[:, :128], 0.0)
    h1 = jnp.maximum((s_ref[1] + xp_ref[1]) * d * gs[:, 128:] + off[:, 128:], 0.0)
    res = jnp.dot(h0, wc_ref[0:128, :], preferred_element_type=jnp.float32)
    res = res + jnp.dot(h1, wc_ref[128:256, :], preferred_element_type=jnp.float32)
    out_ref[...] = res + bc_ref[...]


_R = 1000


def _t0(degp, x, w):
    n = x.shape[0]
    return pl.pallas_call(
        _t0_body,
        grid=(n // _R,),
        in_specs=[
            pl.BlockSpec((2, _R, 128), lambda i: (0, i, 0)),
            pl.BlockSpec((_R, 256), lambda i: (i, 0)),
            pl.BlockSpec((256, 256), lambda i: (0, 0)),
        ],
        out_specs=pl.BlockSpec((2, _R, 128), lambda i: (0, i, 0)),
        out_shape=jax.ShapeDtypeStruct((2, n, 128), jnp.float32),
    )(degp, x, w)


def _tmid(degp, s2, xp2, b, g, be, w):
    n = s2.shape[1]
    return pl.pallas_call(
        _tmid_body,
        grid=(n // _R,),
        in_specs=[
            pl.BlockSpec((2, _R, 128), lambda i: (0, i, 0)),
            pl.BlockSpec((2, _R, 128), lambda i: (0, i, 0)),
            pl.BlockSpec((2, _R, 128), lambda i: (0, i, 0)),
            pl.BlockSpec((1, 256), lambda i: (0, 0)),
            pl.BlockSpec((1, 256), lambda i: (0, 0)),
            pl.BlockSpec((1, 256), lambda i: (0, 0)),
            pl.BlockSpec((256, 256), lambda i: (0, 0)),
        ],
        out_specs=pl.BlockSpec((2, _R, 128), lambda i: (0, i, 0)),
        out_shape=jax.ShapeDtypeStruct((2, n, 128), jnp.float32),
    )(degp, s2, xp2, b.reshape(1, -1), g.reshape(1, -1), be.reshape(1, -1), w)


def _tlast(degp, s2, xp2, b, g, be, wc, bc):
    n = s2.shape[1]
    cdim = wc.shape[1]
    return pl.pallas_call(
        _tlast_body,
        grid=(n // _R,),
        in_specs=[
            pl.BlockSpec((2, _R, 128), lambda i: (0, i, 0)),
            pl.BlockSpec((2, _R, 128), lambda i: (0, i, 0)),
            pl.BlockSpec((2, _R, 128), lambda i: (0, i, 0)),
            pl.BlockSpec((1, 256), lambda i: (0, 0)),
            pl.BlockSpec((1, 256), lambda i: (0, 0)),
            pl.BlockSpec((1, 256), lambda i: (0, 0)),
            pl.BlockSpec((256, cdim), lambda i: (0, 0)),
            pl.BlockSpec((1, cdim), lambda i: (0, 0)),
        ],
        out_specs=pl.BlockSpec((_R, cdim), lambda i: (i, 0)),
        out_shape=jax.ShapeDtypeStruct((n, cdim), jnp.float32),
    )(degp, s2, xp2, b.reshape(1, -1), g.reshape(1, -1), be.reshape(1, -1), wc,
      bc.reshape(1, -1))


def kernel(x, edge_index, W0, b0, g0, be0, W1, b1, g1, be1, W2, b2, g2, be2, Wc, bc):
    n = x.shape[0]
    row = edge_index[0]
    col = edge_index[1]
    rows2 = jnp.concatenate([row, row + n])

    degp = _sc_degree(col, n)

    xws0 = _t0(degp, x, W0)
    s0 = _sc_scatter(xws0.reshape(2 * n, 128), rows2, col, n)
    xws1 = _tmid(degp, s0, xws0, b0, g0, be0, W1)
    s1 = _sc_scatter(xws1.reshape(2 * n, 128), rows2, col, n)
    xws2 = _tmid(degp, s1, xws1, b1, g1, be1, W2)
    s2 = _sc_scatter(xws2.reshape(2 * n, 128), rows2, col, n)
    return _tlast(degp, s2, xws2, b2, g2, be2, Wc, bc)


# speedup vs baseline: 1.0115x; 1.0115x over previous
import functools
import math

import jax
import jax.numpy as jnp
from jax import lax
from jax.experimental import pallas as pl
from jax.experimental.pallas import tpu as pltpu
from jax.experimental.pallas import tpu_sc as plsc

EPS_BN = 1e-5
GSCALE = 1.0 / math.sqrt(1.0 + EPS_BN)

NSUB = 16
NCORE = 2


def _sc_mesh():
    return plsc.VectorSubcoreMesh(core_axis_name="c", subcore_axis_name="s")


def _sc_degree(col, n):
    e = col.shape[0]
    epc = e // NCORE
    ch = 128
    per = (epc // NSUB) & ~(ch - 1)
    m_lo = per // ch
    m_hi = (epc - per * (NSUB - 1)) // ch
    npt = (n // NSUB) & ~7
    tail = n - npt * NSUB

    zeros_h = jnp.zeros((n, 128), jnp.float32)
    ones_h = jnp.ones((ch, 128), jnp.float32)

    @functools.partial(
        pl.kernel,
        out_type=jax.ShapeDtypeStruct((NCORE, n, 128), jnp.float32),
        mesh=_sc_mesh(),
        scratch_types=[
            pltpu.VMEM((4, ch), jnp.int32),
            pltpu.VMEM((ch, 128), jnp.float32),
            pltpu.VMEM_SHARED((n, 128), jnp.float32),
        ] + [pltpu.SemaphoreType.DMA] * 4,
    )
    def k(col_hbm, zero_hbm, ones_hbm, out_hbm, colv, onesv, accum, s0, s1, i0, i1):
        ssem = [s0, s1]
        isem = [i0, i1]
        c = lax.axis_index("c")
        s = lax.axis_index("s")
        m = jnp.where(s == NSUB - 1, m_hi, m_lo)
        base = c * epc + s * per
        pltpu.sync_copy(col_hbm.at[pl.ds(base, ch)], colv.at[0])
        pltpu.sync_copy(col_hbm.at[pl.ds(base + ch, ch)], colv.at[1])
        pltpu.sync_copy(zero_hbm.at[pl.ds(s * npt, npt)], accum.at[pl.ds(s * npt, npt)])

        @pl.when(s == 0)
        def _():
            pltpu.sync_copy(zero_hbm.at[pl.ds(npt * NSUB, tail)],
                            accum.at[pl.ds(npt * NSUB, tail)])

        pltpu.sync_copy(ones_hbm, onesv)
        plsc.subcore_barrier()

        @pl.loop(0, m_hi, step=2)
        def _(i):
            for b in range(2):
                j = i + b

                @pl.when(j < m)
                def _(j=j, b=b):
                    @pl.when(j >= 2)
                    def _():
                        pltpu.make_async_copy(
                            onesv, accum.at[colv.at[j & 3]], ssem[b]).wait()
                        pltpu.make_async_copy(
                            col_hbm.at[pl.ds(0, ch)], colv.at[j & 3],
                            isem[b]).wait()
                    pltpu.async_copy(onesv, accum.at[colv.at[j & 3]],
                                     ssem[b], add=True)

                @pl.when(j + 2 < m)
                def _(j=j, b=b):
                    pltpu.async_copy(
                        col_hbm.at[pl.ds(base + (j + 2) * ch, ch)],
                        colv.at[(j + 2) & 3], isem[b])

        pltpu.make_async_copy(onesv, accum.at[colv.at[0]], ssem[0]).wait()
        pltpu.make_async_copy(onesv, accum.at[colv.at[1]], ssem[1]).wait()
        plsc.subcore_barrier()
        pltpu.sync_copy(accum.at[pl.ds(s * npt, npt)], out_hbm.at[c, pl.ds(s * npt, npt)])

        @pl.when(s == 0)
        def _():
            pltpu.sync_copy(accum.at[pl.ds(npt * NSUB, tail)],
                            out_hbm.at[c, pl.ds(npt * NSUB, tail)])

    return k(col, zeros_h, ones_h)


_NBUF = 2


def _sc_scatter(xws_flat, rows2, col, n):
    e = col.shape[0]
    ch = 128
    per = (e // NSUB) & ~(ch - 1)
    m_lo = per // ch
    m_hi = (e - per * (NSUB - 1)) // ch
    npt = (n // NSUB) & ~7
    tail = n - npt * NSUB

    @functools.partial(
        pl.kernel,
        out_type=jax.ShapeDtypeStruct((NCORE, n, 128), jnp.float32),
        mesh=_sc_mesh(),
        scratch_types=[
            pltpu.VMEM((4, ch), jnp.int32),
            pltpu.VMEM((4, ch), jnp.int32),
            pltpu.VMEM((3, ch, 128), jnp.float32),
            pltpu.VMEM_SHARED((n, 128), jnp.float32),
        ] + [pltpu.SemaphoreType.DMA] * 9,
    )
    def k(xws_hbm, rows_hbm, col_hbm, out_hbm, rowv, colv, msgv, accum,
          g0, g1, g2, s0, s1, s2, i0, i1, i2):
        gsem = [g0, g1, g2]
        ssem = [s0, s1, s2]
        isem = [i0, i1, i2]
        c = lax.axis_index("c")
        s = lax.axis_index("s")
        m = jnp.where(s == NSUB - 1, m_hi, m_lo)
        base = s * per

        def load_idx_sync(j, q):
            pltpu.sync_copy(rows_hbm.at[pl.ds(c * e + base + j * ch, ch)], rowv.at[q])
            pltpu.sync_copy(col_hbm.at[pl.ds(base + j * ch, ch)], colv.at[q])

        load_idx_sync(0, 0)
        load_idx_sync(1, 1)
        load_idx_sync(2, 2)
        pltpu.async_copy(xws_hbm.at[rowv.at[0]], msgv.at[0], gsem[0])
        pltpu.async_copy(xws_hbm.at[rowv.at[1]], msgv.at[1], gsem[1])

        pltpu.sync_copy(xws_hbm.at[pl.ds(c * n + s * npt, npt)],
                        accum.at[pl.ds(s * npt, npt)])

        @pl.when(s == 0)
        def _():
            pltpu.sync_copy(xws_hbm.at[pl.ds(c * n + npt * NSUB, tail)],
                            accum.at[pl.ds(npt * NSUB, tail)])

        plsc.subcore_barrier()

        @pl.loop(0, m_hi + (3 - m_hi % 3) % 3, step=3)
        def _(i):
            for b in range(3):
                j = i + b
                b2 = (b + 2) % 3

                @pl.when(j < m)
                def _(j=j, b=b):
                    pltpu.make_async_copy(
                        xws_hbm.at[rowv.at[j & 3]], msgv.at[b], gsem[b]).wait()
                    pltpu.async_copy(
                        msgv.at[b], accum.at[colv.at[j & 3]], ssem[b], add=True)

                @pl.when(j + 2 < m)
                def _(j=j, b2=b2):
                    @pl.when(j >= 1)
                    def _():
                        pltpu.make_async_copy(
                            msgv.at[b2], accum.at[colv.at[(j - 1) & 3]],
                            ssem[b2]).wait()
                        pltpu.make_async_copy(
                            rows_hbm.at[pl.ds(0, ch)], rowv.at[(j + 2) & 3],
                            isem[b2]).wait()
                        pltpu.make_async_copy(
                            col_hbm.at[pl.ds(0, ch)], colv.at[(j + 2) & 3],
                            isem[b2]).wait()
                    pltpu.async_copy(
                        xws_hbm.at[rowv.at[(j + 2) & 3]], msgv.at[b2], gsem[b2])

                @pl.when(j + 3 < m)
                def _(j=j, b=b):
                    q = (j + 3) & 3
                    pltpu.async_copy(
                        rows_hbm.at[pl.ds(c * e + base + (j + 3) * ch, ch)],
                        rowv.at[q], isem[b])
                    pltpu.async_copy(
                        col_hbm.at[pl.ds(base + (j + 3) * ch, ch)],
                        colv.at[q], isem[b])

        pltpu.make_async_copy(msgv.at[0], accum.at[colv.at[0]], ssem[0]).wait()
        pltpu.make_async_copy(msgv.at[1], accum.at[colv.at[1]], ssem[1]).wait()
        pltpu.make_async_copy(msgv.at[2], accum.at[colv.at[2]], ssem[2]).wait()
        plsc.subcore_barrier()
        pltpu.sync_copy(accum.at[pl.ds(s * npt, npt)], out_hbm.at[c, pl.ds(s * npt, npt)])

        @pl.when(s == 0)
        def _():
            pltpu.sync_copy(accum.at[pl.ds(npt * NSUB, tail)],
                            out_hbm.at[c, pl.ds(npt * NSUB, tail)])

    return k(xws_flat, rows2, col)


def _dinv_block(deg_ref):
    d = deg_ref[0, :, 0:1] + deg_ref[1, :, 0:1] + 1.0
    return lax.rsqrt(d)


def _t0_body(deg_ref, x_ref, w_ref, out_ref):
    d = _dinv_block(deg_ref)
    xw = jnp.dot(x_ref[...], w_ref[...], preferred_element_type=jnp.float32)
    xws = xw * d
    out_ref[0] = xws[:, :128]
    out_ref[1] = xws[:, 128:]


def _tmid_body(deg_ref, s_ref, b_ref, g_ref, be_ref, w_ref, out_ref):
    d = _dinv_block(deg_ref)
    gs = g_ref[...] * GSCALE
    off = b_ref[...] * gs + be_ref[...]
    h0 = jnp.maximum(s_ref[0] * d * gs[:, :128] + off[:, :128], 0.0)
    h1 = jnp.maximum(s_ref[1] * d * gs[:, 128:] + off[:, 128:], 0.0)
    res = jnp.dot(h0, w_ref[0:128, :], preferred_element_type=jnp.float32)
    res = res + jnp.dot(h1, w_ref[128:256, :], preferred_element_type=jnp.float32)
    xws = res * d
    out_ref[0] = xws[:, :128]
    out_ref[1] = xws[:, 128:]


def _tlast_body(deg_ref, s_ref, b_ref, g_ref, be_ref, wc_ref, bc_ref, out_ref):
    d = _dinv_block(deg_ref)
    gs = g_ref[...] * GSCALE
    off = b_ref[...] * gs + be_ref[...]
    h0 = jnp.maximum(s_ref[0] * d * gs[:, :128] + off[:, :128], 0.0)
    h1 = jnp.maximum(s_ref[1] * d * gs[:, 128:] + off[:, 128:], 0.0)
    res = jnp.dot(h0, wc_ref[0:128, :], preferred_element_type=jnp.float32)
    res = res + jnp.dot(h1, wc_ref[128:256, :], preferred_element_type=jnp.float32)
    out_ref[...] = res + bc_ref[...]


_R = 1000


def _t0(degp, x, w):
    n = x.shape[0]
    return pl.pallas_call(
        _t0_body,
        grid=(n // _R,),
        in_specs=[
            pl.BlockSpec((2, _R, 128), lambda i: (0, i, 0)),
            pl.BlockSpec((_R, 256), lambda i: (i, 0)),
            pl.BlockSpec((256, 256), lambda i: (0, 0)),
        ],
        out_specs=pl.BlockSpec((2, _R, 128), lambda i: (0, i, 0)),
        out_shape=jax.ShapeDtypeStruct((2, n, 128), jnp.float32),
    )(degp, x, w)


def _tmid(degp, s2, b, g, be, w):
    n = s2.shape[1]
    return pl.pallas_call(
        _tmid_body,
        grid=(n // _R,),
        in_specs=[
            pl.BlockSpec((2, _R, 128), lambda i: (0, i, 0)),
            pl.BlockSpec((2, _R, 128), lambda i: (0, i, 0)),
            pl.BlockSpec((1, 256), lambda i: (0, 0)),
            pl.BlockSpec((1, 256), lambda i: (0, 0)),
            pl.BlockSpec((1, 256), lambda i: (0, 0)),
            pl.BlockSpec((256, 256), lambda i: (0, 0)),
        ],
        out_specs=pl.BlockSpec((2, _R, 128), lambda i: (0, i, 0)),
        out_shape=jax.ShapeDtypeStruct((2, n, 128), jnp.float32),
    )(degp, s2, b.reshape(1, -1), g.reshape(1, -1), be.reshape(1, -1), w)


def _tlast(degp, s2, b, g, be, wc, bc):
    n = s2.shape[1]
    cdim = wc.shape[1]
    return pl.pallas_call(
        _tlast_body,
        grid=(n // _R,),
        in_specs=[
            pl.BlockSpec((2, _R, 128), lambda i: (0, i, 0)),
            pl.BlockSpec((2, _R, 128), lambda i: (0, i, 0)),
            pl.BlockSpec((1, 256), lambda i: (0, 0)),
            pl.BlockSpec((1, 256), lambda i: (0, 0)),
            pl.BlockSpec((1, 256), lambda i: (0, 0)),
            pl.BlockSpec((256, cdim), lambda i: (0, 0)),
            pl.BlockSpec((1, cdim), lambda i: (0, 0)),
        ],
        out_specs=pl.BlockSpec((_R, cdim), lambda i: (i, 0)),
        out_shape=jax.ShapeDtypeStruct((n, cdim), jnp.float32),
    )(degp, s2, b.reshape(1, -1), g.reshape(1, -1), be.reshape(1, -1), wc,
      bc.reshape(1, -1))


def kernel(x, edge_index, W0, b0, g0, be0, W1, b1, g1, be1, W2, b2, g2, be2, Wc, bc):
    n = x.shape[0]
    row = edge_index[0]
    col = edge_index[1]
    rows2 = jnp.concatenate([row, row + n])

    degp = _sc_degree(col, n)

    xws0 = _t0(degp, x, W0)
    s0 = _sc_scatter(xws0.reshape(2 * n, 128), rows2, col, n)
    xws1 = _tmid(degp, s0, b0, g0, be0, W1)
    s1 = _sc_scatter(xws1.reshape(2 * n, 128), rows2, col, n)
    xws2 = _tmid(degp, s1, b1, g1, be1, W2)
    s2 = _sc_scatter(xws2.reshape(2 * n, 128), rows2, col, n)
    return _tlast(degp, s2, b2, g2, be2, Wc, bc)


# speedup vs baseline: 1.1138x; 1.1012x over previous
import dataclasses
import functools
import math

import jax
import jax.numpy as jnp
from jax import lax
from jax.experimental import pallas as pl
from jax.experimental.pallas import tpu as pltpu
from jax.experimental.pallas import tpu_sc as plsc

EPS_BN = 1e-5
GSCALE = 1.0 / math.sqrt(1.0 + EPS_BN)

NSUB = 16
NCORE = 2


def _sc_mesh():
    return plsc.VectorSubcoreMesh(core_axis_name="c", subcore_axis_name="s")


def _sc_degree(col, n):
    e = col.shape[0]
    epc = e // NCORE
    per = (epc // NSUB) & ~15
    ext = epc - per * NSUB
    m_lo = per // 16
    m_hi = (per + ext) // 16
    nr = -(-n // 128)
    nr = -(-nr // 8) * 8
    zr = nr // 8

    ar_h = jnp.arange(nr, dtype=jnp.int32)

    cp = pltpu.CompilerParams()
    if "needs_layout_passes" in pltpu.CompilerParams.__dataclass_fields__:
        cp = dataclasses.replace(cp, needs_layout_passes=False)

    @functools.partial(
        pl.kernel,
        out_type=jax.ShapeDtypeStruct((NCORE, nr, 128), jnp.float32),
        mesh=_sc_mesh(),
        compiler_params=cp,
        scratch_types=[
            pltpu.VMEM((per + ext,), jnp.int32),
            pltpu.VMEM((nr, 128), jnp.float32),
            pltpu.VMEM((nr,), jnp.int32),
            pltpu.VMEM((8, 128), jnp.float32),
            pltpu.VMEM_SHARED((nr, 128), jnp.float32),
        ],
    )
    def k(col_hbm, ar_hbm, out_hbm, colsv, histv, arv, zbuf, accum):
        c = lax.axis_index("c")
        s = lax.axis_index("s")
        base = c * epc + s * per
        pltpu.sync_copy(col_hbm.at[pl.ds(base, per)], colsv.at[pl.ds(0, per)])

        @pl.when(s == NSUB - 1)
        def _():
            pltpu.sync_copy(col_hbm.at[pl.ds(base + per, ext)],
                            colsv.at[pl.ds(per, ext)])

        pltpu.sync_copy(ar_hbm, arv)

        @pl.loop(0, nr)
        def _(r):
            @pl.loop(0, 128, step=16)
            def _(i):
                histv[r, pl.ds(i, 16)] = jnp.zeros((16,), jnp.float32)

        @pl.loop(0, 8)
        def _(r):
            @pl.loop(0, 128, step=16)
            def _(i):
                zbuf[r, pl.ds(i, 16)] = jnp.zeros((16,), jnp.float32)

        @pl.when(s < zr)
        def _():
            pltpu.sync_copy(zbuf, accum.at[pl.ds(s * 8, 8)])

        m = jnp.where(s == NSUB - 1, m_hi, m_lo)
        ones16 = jnp.ones((16,), jnp.float32)

        @pl.loop(0, m_hi)
        def _(k2):
            @pl.when(k2 < m)
            def _():
                ii = colsv[pl.ds(k2 * 16, 16)]
                rr = lax.shift_right_logical(ii, 7)
                ll = jnp.bitwise_and(ii, 127)
                plsc.addupdate_scatter(histv, [rr, ll], ones16)

        plsc.subcore_barrier()
        pltpu.sync_copy(histv, accum.at[arv], add=True)
        plsc.subcore_barrier()

        @pl.when(s < zr)
        def _():
            pltpu.sync_copy(accum.at[pl.ds(s * 8, 8)],
                            out_hbm.at[c, pl.ds(s * 8, 8)])

    return k(col, ar_h)


_NBUF = 2


def _sc_scatter(xws_flat, rows2, col, n):
    e = col.shape[0]
    ch = 128
    per = (e // NSUB) & ~(ch - 1)
    m_lo = per // ch
    m_hi = (e - per * (NSUB - 1)) // ch
    npt = (n // NSUB) & ~7
    tail = n - npt * NSUB

    @functools.partial(
        pl.kernel,
        out_type=jax.ShapeDtypeStruct((NCORE, n, 128), jnp.float32),
        mesh=_sc_mesh(),
        scratch_types=[
            pltpu.VMEM((4, ch), jnp.int32),
            pltpu.VMEM((4, ch), jnp.int32),
            pltpu.VMEM((3, ch, 128), jnp.float32),
            pltpu.VMEM_SHARED((n, 128), jnp.float32),
        ] + [pltpu.SemaphoreType.DMA] * 9,
    )
    def k(xws_hbm, rows_hbm, col_hbm, out_hbm, rowv, colv, msgv, accum,
          g0, g1, g2, s0, s1, s2, i0, i1, i2):
        gsem = [g0, g1, g2]
        ssem = [s0, s1, s2]
        isem = [i0, i1, i2]
        c = lax.axis_index("c")
        s = lax.axis_index("s")
        m = jnp.where(s == NSUB - 1, m_hi, m_lo)
        base = s * per

        def load_idx_sync(j, q):
            pltpu.sync_copy(rows_hbm.at[pl.ds(c * e + base + j * ch, ch)], rowv.at[q])
            pltpu.sync_copy(col_hbm.at[pl.ds(base + j * ch, ch)], colv.at[q])

        load_idx_sync(0, 0)
        load_idx_sync(1, 1)
        load_idx_sync(2, 2)
        pltpu.async_copy(xws_hbm.at[rowv.at[0]], msgv.at[0], gsem[0])
        pltpu.async_copy(xws_hbm.at[rowv.at[1]], msgv.at[1], gsem[1])

        pltpu.sync_copy(xws_hbm.at[pl.ds(c * n + s * npt, npt)],
                        accum.at[pl.ds(s * npt, npt)])

        @pl.when(s == 0)
        def _():
            pltpu.sync_copy(xws_hbm.at[pl.ds(c * n + npt * NSUB, tail)],
                            accum.at[pl.ds(npt * NSUB, tail)])

        plsc.subcore_barrier()

        @pl.loop(0, m_hi + (3 - m_hi % 3) % 3, step=3)
        def _(i):
            for b in range(3):
                j = i + b
                b2 = (b + 2) % 3

                @pl.when(j < m)
                def _(j=j, b=b):
                    pltpu.make_async_copy(
                        xws_hbm.at[rowv.at[j & 3]], msgv.at[b], gsem[b]).wait()
                    pltpu.async_copy(
                        msgv.at[b], accum.at[colv.at[j & 3]], ssem[b], add=True)

                @pl.when(j + 2 < m)
                def _(j=j, b2=b2):
                    @pl.when(j >= 1)
                    def _():
                        pltpu.make_async_copy(
                            msgv.at[b2], accum.at[colv.at[(j - 1) & 3]],
                            ssem[b2]).wait()
                        pltpu.make_async_copy(
                            rows_hbm.at[pl.ds(0, ch)], rowv.at[(j + 2) & 3],
                            isem[b2]).wait()
                        pltpu.make_async_copy(
                            col_hbm.at[pl.ds(0, ch)], colv.at[(j + 2) & 3],
                            isem[b2]).wait()
                    pltpu.async_copy(
                        xws_hbm.at[rowv.at[(j + 2) & 3]], msgv.at[b2], gsem[b2])

                @pl.when(j + 3 < m)
                def _(j=j, b=b):
                    q = (j + 3) & 3
                    pltpu.async_copy(
                        rows_hbm.at[pl.ds(c * e + base + (j + 3) * ch, ch)],
                        rowv.at[q], isem[b])
                    pltpu.async_copy(
                        col_hbm.at[pl.ds(base + (j + 3) * ch, ch)],
                        colv.at[q], isem[b])

        pltpu.make_async_copy(msgv.at[0], accum.at[colv.at[0]], ssem[0]).wait()
        pltpu.make_async_copy(msgv.at[1], accum.at[colv.at[1]], ssem[1]).wait()
        pltpu.make_async_copy(msgv.at[2], accum.at[colv.at[2]], ssem[2]).wait()
        plsc.subcore_barrier()
        pltpu.sync_copy(accum.at[pl.ds(s * npt, npt)], out_hbm.at[c, pl.ds(s * npt, npt)])

        @pl.when(s == 0)
        def _():
            pltpu.sync_copy(accum.at[pl.ds(npt * NSUB, tail)],
                            out_hbm.at[c, pl.ds(npt * NSUB, tail)])

    return k(xws_flat, rows2, col)


def _dinv_block(deg_ref):
    d = deg_ref[:, 0:1] + deg_ref[:, 1:2] + 1.0
    return lax.rsqrt(d)


def _t0_body(deg_ref, x_ref, w_ref, out_ref):
    d = _dinv_block(deg_ref)
    xw = jnp.dot(x_ref[...], w_ref[...], preferred_element_type=jnp.float32)
    xws = xw * d
    out_ref[0] = xws[:, :128]
    out_ref[1] = xws[:, 128:]


def _tmid_body(deg_ref, s_ref, b_ref, g_ref, be_ref, w_ref, out_ref):
    d = _dinv_block(deg_ref)
    gs = g_ref[...] * GSCALE
    off = b_ref[...] * gs + be_ref[...]
    h0 = jnp.maximum(s_ref[0] * d * gs[:, :128] + off[:, :128], 0.0)
    h1 = jnp.maximum(s_ref[1] * d * gs[:, 128:] + off[:, 128:], 0.0)
    res = jnp.dot(h0, w_ref[0:128, :], preferred_element_type=jnp.float32)
    res = res + jnp.dot(h1, w_ref[128:256, :], preferred_element_type=jnp.float32)
    xws = res * d
    out_ref[0] = xws[:, :128]
    out_ref[1] = xws[:, 128:]


def _tlast_body(deg_ref, s_ref, b_ref, g_ref, be_ref, wc_ref, bc_ref, out_ref):
    d = _dinv_block(deg_ref)
    gs = g_ref[...] * GSCALE
    off = b_ref[...] * gs + be_ref[...]
    h0 = jnp.maximum(s_ref[0] * d * gs[:, :128] + off[:, :128], 0.0)
    h1 = jnp.maximum(s_ref[1] * d * gs[:, 128:] + off[:, 128:], 0.0)
    res = jnp.dot(h0, wc_ref[0:128, :], preferred_element_type=jnp.float32)
    res = res + jnp.dot(h1, wc_ref[128:256, :], preferred_element_type=jnp.float32)
    out_ref[...] = res + bc_ref[...]


_R = 1000


def _t0(degp, x, w):
    n = x.shape[0]
    return pl.pallas_call(
        _t0_body,
        grid=(n // _R,),
        in_specs=[
            pl.BlockSpec((_R, 2), lambda i: (i, 0)),
            pl.BlockSpec((_R, 256), lambda i: (i, 0)),
            pl.BlockSpec((256, 256), lambda i: (0, 0)),
        ],
        out_specs=pl.BlockSpec((2, _R, 128), lambda i: (0, i, 0)),
        out_shape=jax.ShapeDtypeStruct((2, n, 128), jnp.float32),
    )(degp, x, w)


def _tmid(degp, s2, b, g, be, w):
    n = s2.shape[1]
    return pl.pallas_call(
        _tmid_body,
        grid=(n // _R,),
        in_specs=[
            pl.BlockSpec((_R, 2), lambda i: (i, 0)),
            pl.BlockSpec((2, _R, 128), lambda i: (0, i, 0)),
            pl.BlockSpec((1, 256), lambda i: (0, 0)),
            pl.BlockSpec((1, 256), lambda i: (0, 0)),
            pl.BlockSpec((1, 256), lambda i: (0, 0)),
            pl.BlockSpec((256, 256), lambda i: (0, 0)),
        ],
        out_specs=pl.BlockSpec((2, _R, 128), lambda i: (0, i, 0)),
        out_shape=jax.ShapeDtypeStruct((2, n, 128), jnp.float32),
    )(degp, s2, b.reshape(1, -1), g.reshape(1, -1), be.reshape(1, -1), w)


def _tlast(degp, s2, b, g, be, wc, bc):
    n = s2.shape[1]
    cdim = wc.shape[1]
    return pl.pallas_call(
        _tlast_body,
        grid=(n // _R,),
        in_specs=[
            pl.BlockSpec((_R, 2), lambda i: (i, 0)),
            pl.BlockSpec((2, _R, 128), lambda i: (0, i, 0)),
            pl.BlockSpec((1, 256), lambda i: (0, 0)),
            pl.BlockSpec((1, 256), lambda i: (0, 0)),
            pl.BlockSpec((1, 256), lambda i: (0, 0)),
            pl.BlockSpec((256, cdim), lambda i: (0, 0)),
            pl.BlockSpec((1, cdim), lambda i: (0, 0)),
        ],
        out_specs=pl.BlockSpec((_R, cdim), lambda i: (i, 0)),
        out_shape=jax.ShapeDtypeStruct((n, cdim), jnp.float32),
    )(degp, s2, b.reshape(1, -1), g.reshape(1, -1), be.reshape(1, -1), wc,
      bc.reshape(1, -1))


def kernel(x, edge_index, W0, b0, g0, be0, W1, b1, g1, be1, W2, b2, g2, be2, Wc, bc):
    n = x.shape[0]
    row = edge_index[0]
    col = edge_index[1]
    rows2 = jnp.concatenate([row, row + n])

    degh = _sc_degree(col, n)
    degp = degh.reshape(2, -1)[:, :n].T

    xws0 = _t0(degp, x, W0)
    s0 = _sc_scatter(xws0.reshape(2 * n, 128), rows2, col, n)
    xws1 = _tmid(degp, s0, b0, g0, be0, W1)
    s1 = _sc_scatter(xws1.reshape(2 * n, 128), rows2, col, n)
    xws2 = _tmid(degp, s1, b1, g1, be1, W2)
    s2 = _sc_scatter(xws2.reshape(2 * n, 128), rows2, col, n)
    return _tlast(degp, s2, b2, g2, be2, Wc, bc)
